# baseline (speedup 1.0000x reference)
import functools
import math

import jax
import jax.numpy as jnp
from jax import lax
from jax.experimental import pallas as pl
from jax.experimental.pallas import tpu as pltpu
from jax.experimental.pallas import tpu_sc as plsc

BATCH = 4096
HIST = 200
D = 64
B = BATCH * HIST
NC, NS = 2, 16
NW = NC * NS
ROWS_PER_W = B // NW
CHUNK = 128
NCHUNK = ROWS_PER_W // CHUNK
CPP = 8
NPLANE = NCHUNK // CPP
NBUF = 5
NPB = 2
SCALE = math.sqrt(D)

_mesh = plsc.VectorSubcoreMesh(
    core_axis_name="c", subcore_axis_name="s", num_cores=NC, num_subcores=NS
)


@functools.partial(
    pl.kernel,
    mesh=_mesh,
    out_type=jax.ShapeDtypeStruct((B, D), jnp.float32),
    scratch_types=[
        pltpu.VMEM((NPB, 8, 128), jnp.int32),
        pltpu.VMEM((NBUF, CHUNK, 128), jnp.float32),
        pltpu.VMEM((2, CHUNK, D), jnp.float32),
        pltpu.SemaphoreType.DMA((NBUF,)),
        pltpu.SemaphoreType.DMA,
        pltpu.SemaphoreType.DMA,
    ],
    compiler_params=pltpu.CompilerParams(needs_layout_passes=False),
)
def _embed_sc(table2_hbm, src_hbm, out_hbm, idx_v, rows_v, outc_v,
              gsem, o0, o1):
    wid = lax.axis_index("s") * NC + lax.axis_index("c")
    plane0 = wid * NPLANE
    base = wid * ROWS_PER_W
    osems = [o0, o1]

    def load_plane(p):
        pltpu.sync_copy(src_hbm.at[plane0 + p], idx_v.at[p % NPB])

    def fire_gather(i, krow, b):
        pb = (i // CPP) % NPB
        pltpu.async_copy(
            table2_hbm.at[idx_v.at[pb, krow]], rows_v.at[b], gsem.at[b]
        )

    def drain_gather(b):
        pltpu.make_async_copy(
            table2_hbm.at[pl.ds(0, CHUNK)], rows_v.at[b], gsem.at[b]
        ).wait()

    def drain_out(b):
        pltpu.make_async_copy(
            outc_v.at[b], out_hbm.at[pl.ds(base, CHUNK)], osems[b]
        ).wait()

    load_plane(0)
    fire_gather(0, 0, 0)
    fire_gather(1, 1, 1)
    fire_gather(2, 2, 2)

    def plane_body(p, carry):
        @pl.when(p < NPLANE - 1)
        def _():
            load_plane(p + 1)

        i0 = p * CPP
        for k in range(CPP):
            i = i0 + k
            b = i % NBUF
            ob = k % 2
            pb = (i // CPP) % NPB
            drain_gather(b)
            if k < 2:
                @pl.when(p > 0)
                def _():
                    drain_out(ob)
            else:
                drain_out(ob)

            def sel_row(r, c2, b=b, ob=ob):
                for j in range(D // 16):
                    sl = pl.ds(j * 16, 16)
                    outc_v[ob, r, sl] = rows_v[b, r, sl] * SCALE
                return c2

            lax.fori_loop(0, CHUNK, sel_row, 0, unroll=4)
            pltpu.async_copy(
                outc_v.at[ob],
                out_hbm.at[pl.ds(base + i * CHUNK, CHUNK)],
                osems[ob],
            )
            if k < CPP - 3:
                fire_gather(i + 3, (k + 3) % CPP, (i + 3) % NBUF)
            else:
                @pl.when(p < NPLANE - 1)
                def _(k=k, i=i):
                    fire_gather(i + 3, (k + 3) % CPP, (i + 3) % NBUF)
        return carry

    lax.fori_loop(0, NPLANE, plane_body, 0)
    drain_out(0)
    drain_out(1)


def kernel(source, table):
    src = source.astype(jnp.int32).reshape(B // 1024, 8, 128)
    table2 = jnp.pad(table, ((0, 0), (0, 64)))
    out = _embed_sc(table2, src)
    return out.reshape(BATCH, HIST, D)


# speedup vs baseline: 1.1941x; 1.1941x over previous
import functools
import math

import jax
import jax.numpy as jnp
from jax import lax
from jax.experimental import pallas as pl
from jax.experimental.pallas import tpu as pltpu
from jax.experimental.pallas import tpu_sc as plsc

BATCH = 4096
HIST = 200
D = 64
B = BATCH * HIST
NC, NS = 2, 16
NW = NC * NS
ROWS_PER_W = B // NW
CHUNK = 128
NCHUNK = ROWS_PER_W // CHUNK
CPP = 8
NPLANE = NCHUNK // CPP
NBUF = 4
NPB = 2
SCALE = math.sqrt(D)

_mesh = plsc.VectorSubcoreMesh(
    core_axis_name="c", subcore_axis_name="s", num_cores=NC, num_subcores=NS
)


@functools.partial(
    pl.kernel,
    mesh=_mesh,
    out_type=jax.ShapeDtypeStruct((B, D), jnp.float32),
    scratch_types=[
        pltpu.VMEM((NPB, 8, 128), jnp.int32),
        pltpu.VMEM((NBUF, CHUNK, 128), jnp.float32),
        pltpu.VMEM((2, CHUNK, D), jnp.float32),
        pltpu.SemaphoreType.DMA,
        pltpu.SemaphoreType.DMA,
        pltpu.SemaphoreType.DMA,
        pltpu.SemaphoreType.DMA,
        pltpu.SemaphoreType.DMA,
        pltpu.SemaphoreType.DMA,
    ],
    compiler_params=pltpu.CompilerParams(needs_layout_passes=False),
)
def _embed_sc(table2_hbm, src_hbm, out_hbm, idx_v, rows_v, outc_v,
              g0, g1, g2, g3, o0, o1):
    wid = lax.axis_index("s") * NC + lax.axis_index("c")
    plane0 = wid * NPLANE
    base = wid * ROWS_PER_W
    gsems = [g0, g1, g2, g3]
    osems = [o0, o1]

    def load_plane(p):
        pltpu.sync_copy(src_hbm.at[plane0 + p], idx_v.at[p % NPB])

    def fire_gather(i, krow, b):
        pb = (i // CPP) % NPB
        pltpu.async_copy(
            table2_hbm.at[idx_v.at[pb, krow]], rows_v.at[b], gsems[b]
        )

    def drain_gather(b):
        pltpu.make_async_copy(
            table2_hbm.at[pl.ds(0, CHUNK)], rows_v.at[b], gsems[b]
        ).wait()

    def drain_out(b):
        pltpu.make_async_copy(
            outc_v.at[b], out_hbm.at[pl.ds(base, CHUNK)], osems[b]
        ).wait()

    load_plane(0)
    fire_gather(0, 0, 0)
    fire_gather(1, 1, 1)

    def plane_body(p, carry):
        @pl.when(p < NPLANE - 1)
        def _():
            load_plane(p + 1)

        i0 = p * CPP
        for k in range(CPP):
            i = i0 + k
            b = k % NBUF
            ob = k % 2
            pb = (i // CPP) % NPB
            drain_gather(b)
            if k < 2:
                @pl.when(p > 0)
                def _():
                    drain_out(ob)
            else:
                drain_out(ob)

            def sel_row(r, c2, b=b, ob=ob):
                for j in range(D // 16):
                    sl = pl.ds(j * 16, 16)
                    outc_v[ob, r, sl] = rows_v[b, r, sl] * SCALE
                return c2

            lax.fori_loop(0, CHUNK, sel_row, 0, unroll=4)
            pltpu.async_copy(
                outc_v.at[ob],
                out_hbm.at[pl.ds(base + i * CHUNK, CHUNK)],
                osems[ob],
            )
            if k < CPP - 2:
                fire_gather(i + 2, (k + 2) % CPP, (k + 2) % NBUF)
            else:
                @pl.when(p < NPLANE - 1)
                def _(k=k, i=i):
                    fire_gather(i + 2, (k + 2) % CPP, (k + 2) % NBUF)
        return carry

    lax.fori_loop(0, NPLANE, plane_body, 0)
    drain_out(0)
    drain_out(1)


def kernel(source, table):
    src = source.astype(jnp.int32).reshape(B // 1024, 8, 128)
    table2 = jnp.pad(table, ((0, 0), (0, 64)))
    out = _embed_sc(table2, src)
    return out.reshape(BATCH, HIST, D)


# speedup vs baseline: 1.2341x; 1.0335x over previous
import functools
import math

import jax
import jax.numpy as jnp
from jax import lax
from jax.experimental import pallas as pl
from jax.experimental.pallas import tpu as pltpu
from jax.experimental.pallas import tpu_sc as plsc

BATCH = 4096
HIST = 200
D = 64
B = BATCH * HIST
NC, NS = 2, 16
NW = NC * NS
ROWS_PER_W = B // NW
CHUNK = 128
NCHUNK = ROWS_PER_W // CHUNK
CPP = 8
NPLANE = NCHUNK // CPP
NBUF = 4
NPB = 2
SCALE = math.sqrt(D)

_mesh = plsc.VectorSubcoreMesh(
    core_axis_name="c", subcore_axis_name="s", num_cores=NC, num_subcores=NS
)


@functools.partial(
    pl.kernel,
    mesh=_mesh,
    out_type=jax.ShapeDtypeStruct((B, D), jnp.float32),
    scratch_types=[
        pltpu.VMEM((NPB, 8, 128), jnp.int32),
        pltpu.VMEM((NBUF, CHUNK, 128), jnp.float32),
        pltpu.VMEM((2, CHUNK, D), jnp.float32),
        pltpu.SemaphoreType.DMA,
        pltpu.SemaphoreType.DMA,
        pltpu.SemaphoreType.DMA,
        pltpu.SemaphoreType.DMA,
        pltpu.SemaphoreType.DMA,
        pltpu.SemaphoreType.DMA,
    ],
    compiler_params=pltpu.CompilerParams(needs_layout_passes=False),
)
def _embed_sc(table2_hbm, src_hbm, out_hbm, idx_v, rows_v, outc_v,
              g0, g1, g2, g3, o0, o1):
    wid = lax.axis_index("s") * NC + lax.axis_index("c")
    plane0 = wid * NPLANE
    base = wid * ROWS_PER_W
    gsems = [g0, g1, g2, g3]
    osems = [o0, o1]

    def load_plane(p):
        pltpu.sync_copy(src_hbm.at[plane0 + p], idx_v.at[p % NPB])

    def fire_gather(i, krow, b):
        pb = (i // CPP) % NPB
        pltpu.async_copy(
            table2_hbm.at[idx_v.at[pb, krow]], rows_v.at[b], gsems[b]
        )

    def drain_gather(b):
        pltpu.make_async_copy(
            table2_hbm.at[pl.ds(0, CHUNK)], rows_v.at[b], gsems[b]
        ).wait()

    def drain_out(b):
        pltpu.make_async_copy(
            outc_v.at[b], out_hbm.at[pl.ds(base, CHUNK)], osems[b]
        ).wait()

    load_plane(0)
    fire_gather(0, 0, 0)
    fire_gather(1, 1, 1)
    fire_gather(2, 2, 2)

    def plane_body(p, carry):
        @pl.when(p < NPLANE - 1)
        def _():
            load_plane(p + 1)

        i0 = p * CPP
        for k in range(CPP):
            i = i0 + k
            b = k % NBUF
            ob = k % 2
            pb = (i // CPP) % NPB
            drain_gather(b)
            if k < 2:
                @pl.when(p > 0)
                def _():
                    drain_out(ob)
            else:
                drain_out(ob)

            def sel_row(r, c2, b=b, ob=ob):
                for j in range(D // 16):
                    sl = pl.ds(j * 16, 16)
                    outc_v[ob, r, sl] = rows_v[b, r, sl] * SCALE
                return c2

            lax.fori_loop(0, CHUNK, sel_row, 0, unroll=8)
            pltpu.async_copy(
                outc_v.at[ob],
                out_hbm.at[pl.ds(base + i * CHUNK, CHUNK)],
                osems[ob],
            )
            if k < CPP - 3:
                fire_gather(i + 3, (k + 3) % CPP, (k + 3) % NBUF)
            else:
                @pl.when(p < NPLANE - 1)
                def _(k=k, i=i):
                    fire_gather(i + 3, (k + 3) % CPP, (k + 3) % NBUF)
        return carry

    lax.fori_loop(0, NPLANE, plane_body, 0)
    drain_out(0)
    drain_out(1)


def kernel(source, table):
    src = source.astype(jnp.int32).reshape(B // 1024, 8, 128)
    table2 = jnp.pad(table, ((0, 0), (0, 64)))
    out = _embed_sc(table2, src)
    return out.reshape(BATCH, HIST, D)


# speedup vs baseline: 1.2453x; 1.0091x over previous
import functools
import math

import jax
import jax.numpy as jnp
from jax import lax
from jax.experimental import pallas as pl
from jax.experimental.pallas import tpu as pltpu
from jax.experimental.pallas import tpu_sc as plsc

BATCH = 4096
HIST = 200
D = 64
B = BATCH * HIST
NC, NS = 2, 16
NW = NC * NS
ROWS_PER_W = B // NW
CHUNK = 128
NCHUNK = ROWS_PER_W // CHUNK
CPP = 8
NPLANE = NCHUNK // CPP
NBUF = 4
NPB = 2
SCALE = math.sqrt(D)

_mesh = plsc.VectorSubcoreMesh(
    core_axis_name="c", subcore_axis_name="s", num_cores=NC, num_subcores=NS
)


@functools.partial(
    pl.kernel,
    mesh=_mesh,
    out_type=jax.ShapeDtypeStruct((B, D), jnp.float32),
    scratch_types=[
        pltpu.VMEM((NPB, 8, 128), jnp.int32),
        pltpu.VMEM((NBUF, CHUNK, 128), jnp.float32),
        pltpu.VMEM((2, CHUNK, D), jnp.float32),
        pltpu.SemaphoreType.DMA,
        pltpu.SemaphoreType.DMA,
        pltpu.SemaphoreType.DMA,
        pltpu.SemaphoreType.DMA,
        pltpu.SemaphoreType.DMA,
        pltpu.SemaphoreType.DMA,
        pltpu.SemaphoreType.DMA,
    ],
    compiler_params=pltpu.CompilerParams(needs_layout_passes=False),
)
def _embed_sc(table2_hbm, src_hbm, out_hbm, idx_v, rows_v, outc_v,
              g0, g1, g2, g3, o0, o1, psem):
    wid = lax.axis_index("s") * NC + lax.axis_index("c")
    plane0 = wid * NPLANE
    base = wid * ROWS_PER_W
    gsems = [g0, g1, g2, g3]
    osems = [o0, o1]

    def load_plane(p):
        pltpu.async_copy(src_hbm.at[plane0 + p], idx_v.at[p % NPB], psem)

    def drain_plane():
        pltpu.make_async_copy(
            src_hbm.at[plane0], idx_v.at[0], psem
        ).wait()

    def fire_gather(i, krow, b):
        pb = (i // CPP) % NPB
        pltpu.async_copy(
            table2_hbm.at[idx_v.at[pb, krow]], rows_v.at[b], gsems[b]
        )

    def drain_gather(b):
        pltpu.make_async_copy(
            table2_hbm.at[pl.ds(0, CHUNK)], rows_v.at[b], gsems[b]
        ).wait()

    def drain_out(b):
        pltpu.make_async_copy(
            outc_v.at[b], out_hbm.at[pl.ds(base, CHUNK)], osems[b]
        ).wait()

    load_plane(0)
    drain_plane()
    fire_gather(0, 0, 0)
    fire_gather(1, 1, 1)
    fire_gather(2, 2, 2)

    def plane_body(p, carry):
        @pl.when(p < NPLANE - 1)
        def _():
            load_plane(p + 1)

        i0 = p * CPP
        for k in range(CPP):
            i = i0 + k
            b = k % NBUF
            ob = k % 2
            pb = (i // CPP) % NPB
            drain_gather(b)
            if k < 2:
                @pl.when(p > 0)
                def _():
                    drain_out(ob)
            else:
                drain_out(ob)

            def sel_row(r, c2, b=b, ob=ob):
                for j in range(D // 16):
                    sl = pl.ds(j * 16, 16)
                    outc_v[ob, r, sl] = rows_v[b, r, sl] * SCALE
                return c2

            lax.fori_loop(0, CHUNK, sel_row, 0, unroll=8)
            pltpu.async_copy(
                outc_v.at[ob],
                out_hbm.at[pl.ds(base + i * CHUNK, CHUNK)],
                osems[ob],
            )
            if k < CPP - 3:
                fire_gather(i + 3, (k + 3) % CPP, (k + 3) % NBUF)
            else:
                @pl.when(p < NPLANE - 1)
                def _(k=k, i=i):
                    if k == CPP - 3:
                        drain_plane()
                    fire_gather(i + 3, (k + 3) % CPP, (k + 3) % NBUF)
        return carry

    lax.fori_loop(0, NPLANE, plane_body, 0)
    drain_out(0)
    drain_out(1)


def kernel(source, table):
    src = source.astype(jnp.int32).reshape(B // 1024, 8, 128)
    table2 = jnp.pad(table, ((0, 0), (0, 64)))
    out = _embed_sc(table2, src)
    return out.reshape(BATCH, HIST, D)
